# Initial kernel scaffold; baseline (speedup 1.0000x reference)
import jax
import jax.numpy as jnp
from jax.experimental import pallas as pl


def kernel(node_input, node_attr, edge_src, edge_dst, edge_attr, edge_scalars, batch, W_q, b_q, W_src, b_src, W_dst, W_fc1, b_fc1, W_fc2, b_fc2, W_fc3, W_kv, b_kv, W_proj, b_proj):
    raise NotImplementedError("write your pallas kernel here")


# speedup vs baseline: 19.3878x; 19.3878x over previous
import functools

import jax
import jax.numpy as jnp
import numpy as np
from jax import lax
from jax.experimental import pallas as pl
from jax.experimental.pallas import tpu as pltpu
from jax.experimental.pallas import tpu_sc as plsc

N = 10000
E = 320000
D = 128
H = 4
DH = 32
DS = 16
HID = 64

NC = 2
NS = 16
NW = NC * NS
EPW = E // NW
EPC = E // NC
CH = 80
G_ITERS = EPW // CH
NP = 10240
NPT = NP // NS

_mesh = plsc.VectorSubcoreMesh(core_axis_name="c", subcore_axis_name="s",
                               num_cores=NC, num_subcores=NS)


def _nodeproj_body(x_ref, w_ref, b_ref, tsrc_ref, tdst_ref, tq_ref):
    p = jnp.dot(x_ref[...], w_ref[...], preferred_element_type=jnp.float32)
    p = p + b_ref[...]
    tsrc_ref[...] = p[:, :D]
    tdst_ref[...] = p[:, D:2 * D]
    tq_ref[...] = p[:, 2 * D:]


def _node_projections(x, w_cat, b_cat):
    blk = 400
    grid = N // blk
    return pl.pallas_call(
        _nodeproj_body,
        grid=(grid,),
        in_specs=[
            pl.BlockSpec((blk, D), lambda i: (i, 0)),
            pl.BlockSpec((D, 3 * D), lambda i: (0, 0)),
            pl.BlockSpec((1, 3 * D), lambda i: (0, 0)),
        ],
        out_specs=[
            pl.BlockSpec((blk, D), lambda i: (i, 0)),
            pl.BlockSpec((blk, D), lambda i: (i, 0)),
            pl.BlockSpec((blk, D), lambda i: (i, 0)),
        ],
        out_shape=[
            jax.ShapeDtypeStruct((N, D), jnp.float32),
            jax.ShapeDtypeStruct((N, D), jnp.float32),
            jax.ShapeDtypeStruct((N, D), jnp.float32),
        ],
    )(x, w_cat, b_cat)


@functools.partial(
    pl.kernel,
    mesh=_mesh,
    out_type=(
        jax.ShapeDtypeStruct((E, D), jnp.float32),
        jax.ShapeDtypeStruct((E, D), jnp.float32),
        jax.ShapeDtypeStruct((E, D), jnp.float32),
    ),
    scratch_types=[
        pltpu.VMEM((CH,), jnp.int32),
        pltpu.VMEM((CH,), jnp.int32),
        pltpu.VMEM((CH, D), jnp.float32),
        pltpu.VMEM((CH, D), jnp.float32),
        pltpu.VMEM((CH, D), jnp.float32),
        pltpu.SemaphoreType.DMA,
    ],
)
def _gather_kernel(tsrc_hbm, tdst_hbm, tq_hbm, esrc_hbm, edst_hbm,
                   gsrc_hbm, gdst_hbm, gq_hbm,
                   idxs_v, idxd_v, rows_s, rows_d, rows_q, sem):
    wid = lax.axis_index("s") * NC + lax.axis_index("c")
    base = wid * EPW

    def body(i, carry):
        off = base + i * CH
        pltpu.sync_copy(esrc_hbm.at[pl.ds(off, CH)], idxs_v)
        pltpu.sync_copy(edst_hbm.at[pl.ds(off, CH)], idxd_v)
        cp_s = pltpu.async_copy(tsrc_hbm.at[idxs_v], rows_s, sem)
        cp_d = pltpu.async_copy(tdst_hbm.at[idxd_v], rows_d, sem)
        cp_q = pltpu.async_copy(tq_hbm.at[idxd_v], rows_q, sem)
        cp_s.wait()
        cp_d.wait()
        cp_q.wait()
        pltpu.sync_copy(rows_s, gsrc_hbm.at[pl.ds(off, CH)])
        pltpu.sync_copy(rows_d, gdst_hbm.at[pl.ds(off, CH)])
        pltpu.sync_copy(rows_q, gq_hbm.at[pl.ds(off, CH)])
        return carry

    lax.fori_loop(0, G_ITERS, body, 0)


def _edge_body(gsrc_ref, gdst_ref, gq_ref, escal_ref, eattr_ref,
               w1_ref, b1_ref, w2_ref, b2_ref, w3_ref,
               wkv_ref, bkv_ref, sel_ref,
               wv_ref, wb_ref):
    x = gsrc_ref[...] + gdst_ref[...]
    q = gq_ref[...]
    h = escal_ref[...] @ w1_ref[...] + b1_ref[...]
    h = h * jax.nn.sigmoid(h)
    h = h @ w2_ref[...] + b2_ref[...]
    h = h * jax.nn.sigmoid(h)
    dtp = jnp.dot(h, w3_ref[...], preferred_element_type=jnp.float32)
    kvp = x * eattr_ref[...] * dtp
    kv = jnp.dot(kvp, wkv_ref[...], preferred_element_type=jnp.float32)
    kv = kv + bkv_ref[...]
    k = kv[:, :D]
    v = kv[:, D:]
    qk = q * k
    alpha = jnp.dot(qk, sel_ref[...], preferred_element_type=jnp.float32)
    w4 = jnp.exp(alpha)
    wb = jnp.dot(w4, sel_ref[...].T,
                 preferred_element_type=jnp.float32)
    wv_ref[...] = v * wb
    wb_ref[...] = wb


def _edge_pipeline(gsrc, gdst, gq, escal, eattr, w1, b1, w2, b2, w3, wkv, bkv,
                   sel):
    blk = 512
    grid = E // blk
    return pl.pallas_call(
        _edge_body,
        grid=(grid,),
        in_specs=[
            pl.BlockSpec((blk, D), lambda i: (i, 0)),
            pl.BlockSpec((blk, D), lambda i: (i, 0)),
            pl.BlockSpec((blk, D), lambda i: (i, 0)),
            pl.BlockSpec((blk, DS), lambda i: (i, 0)),
            pl.BlockSpec((blk, 1), lambda i: (i, 0)),
            pl.BlockSpec((DS, HID), lambda i: (0, 0)),
            pl.BlockSpec((1, HID), lambda i: (0, 0)),
            pl.BlockSpec((HID, HID), lambda i: (0, 0)),
            pl.BlockSpec((1, HID), lambda i: (0, 0)),
            pl.BlockSpec((HID, D), lambda i: (0, 0)),
            pl.BlockSpec((D, 2 * D), lambda i: (0, 0)),
            pl.BlockSpec((1, 2 * D), lambda i: (0, 0)),
            pl.BlockSpec((D, H), lambda i: (0, 0)),
        ],
        out_specs=[
            pl.BlockSpec((blk, D), lambda i: (i, 0)),
            pl.BlockSpec((blk, D), lambda i: (i, 0)),
        ],
        out_shape=[
            jax.ShapeDtypeStruct((E, D), jnp.float32),
            jax.ShapeDtypeStruct((E, D), jnp.float32),
        ],
    )(gsrc, gdst, gq, escal, eattr, w1, b1, w2, b2, w3, wkv, bkv, sel)


@functools.partial(
    pl.kernel,
    mesh=_mesh,
    out_type=(
        jax.ShapeDtypeStruct((NC * NP, D), jnp.float32),
        jax.ShapeDtypeStruct((NC * NP, D), jnp.float32),
    ),
    scratch_types=[
        pltpu.VMEM((CH, D), jnp.float32),
        pltpu.VMEM((CH,), jnp.int32),
        pltpu.VMEM_SHARED((NP, D), jnp.float32),
    ],
)
def _scatter_kernel(wv_hbm, wb_hbm, edst_hbm, onum_hbm, oden_hbm,
                    pay_v, idx_v, stab):
    cid = lax.axis_index("c")
    sid = lax.axis_index("s")
    base = cid * EPC + sid * EPW

    def one_phase(pay_hbm, out_hbm):
        def zrow(i, carry):
            for j in range(D // 16):
                pay_v[i, pl.ds(j * 16, 16)] = jnp.zeros((16,), jnp.float32)
            return carry

        lax.fori_loop(0, CH, zrow, 0)

        def zcopy(j, carry):
            pltpu.sync_copy(pay_v, stab.at[pl.ds(sid * NPT + j * CH, CH)])
            return carry

        lax.fori_loop(0, NPT // CH, zcopy, 0)
        plsc.subcore_barrier()

        def body(i, carry):
            off = base + i * CH
            pltpu.sync_copy(edst_hbm.at[pl.ds(off, CH)], idx_v)
            pltpu.sync_copy(pay_hbm.at[pl.ds(off, CH)], pay_v)
            pltpu.sync_copy(pay_v, stab.at[idx_v], add=True)
            return carry

        lax.fori_loop(0, EPW // CH, body, 0)
        plsc.subcore_barrier()

        pltpu.sync_copy(stab.at[pl.ds(sid * NPT, NPT)],
                        out_hbm.at[pl.ds(cid * NP + sid * NPT, NPT)])
        plsc.subcore_barrier()

    one_phase(wv_hbm, onum_hbm)
    one_phase(wb_hbm, oden_hbm)


def _combine_body(num0_ref, num1_ref, den0_ref, den1_ref,
                  wproj_ref, bproj_ref, out_ref):
    den = den0_ref[...] + den1_ref[...]
    o = (num0_ref[...] + num1_ref[...]) / (den + 1e-16)
    out_ref[...] = jnp.dot(o, wproj_ref[...],
                           preferred_element_type=jnp.float32) + bproj_ref[...]


def _combine(num, den, wproj, bproj):
    blk = 80
    grid = N // blk
    nb = NP // blk
    return pl.pallas_call(
        _combine_body,
        grid=(grid,),
        in_specs=[
            pl.BlockSpec((blk, D), lambda i: (i, 0)),
            pl.BlockSpec((blk, D), lambda i: (i + nb, 0)),
            pl.BlockSpec((blk, D), lambda i: (i, 0)),
            pl.BlockSpec((blk, D), lambda i: (i + nb, 0)),
            pl.BlockSpec((D, D), lambda i: (0, 0)),
            pl.BlockSpec((1, D), lambda i: (0, 0)),
        ],
        out_specs=pl.BlockSpec((blk, D), lambda i: (i, 0)),
        out_shape=jax.ShapeDtypeStruct((N, D), jnp.float32),
    )(num, num, den, den, wproj, bproj)


def kernel(node_input, node_attr, edge_src, edge_dst, edge_attr, edge_scalars,
           batch, W_q, b_q, W_src, b_src, W_dst, W_fc1, b_fc1, W_fc2, b_fc2,
           W_fc3, W_kv, b_kv, W_proj, b_proj):
    scale = np.float32(1.0 / np.sqrt(DH))
    w_cat = jnp.concatenate([W_src, W_dst, W_q * scale], axis=1)
    b_cat = jnp.concatenate(
        [b_src, jnp.zeros((D,), jnp.float32), b_q * scale])[None, :]
    sel = jnp.repeat(jnp.eye(H, dtype=jnp.float32), DH, axis=0)

    tsrc, tdst, tq = _node_projections(node_input, w_cat, b_cat)
    gsrc, gdst, gq = _gather_kernel(tsrc, tdst, tq, edge_src, edge_dst)
    wv, wb = _edge_pipeline(
        gsrc, gdst, gq, edge_scalars, edge_attr,
        W_fc1, b_fc1[None, :], W_fc2, b_fc2[None, :], W_fc3,
        W_kv, b_kv[None, :], sel)
    num, den = _scatter_kernel(wv, wb, edge_dst)
    return _combine(num, den, W_proj, b_proj[None, :])
